# baseline (speedup 1.0000x reference)
import functools

import jax
import jax.numpy as jnp
import numpy as np
from jax import lax
from jax.experimental import pallas as pl
from jax.experimental.pallas import tpu as pltpu
from jax.experimental.pallas import tpu_sc as plsc

N = 10000
E = 320000
D = 128

NC = 2
NS = 16
NW = NC * NS

CHUNK = 50
EPT = E // NW
NCHUNK = EPT // CHUNK
BCH = 20
NBLK = NCHUNK // BCH
ZB = 48
ROWS_PER_TILE = 632
NPAD = ROWS_PER_TILE * NS
CUNROLL = 5

_PERM = np.concatenate(
    [np.stack([np.arange(16) + 32 * b, np.arange(16) + 32 * b + 16],
              axis=1).reshape(-1) for b in range(D // 32)])


def _convert_chunk(gbuf, fbuf):
    mask = jnp.int32(-65536)

    def crow(i, _):
        for u in range(CUNROLL):
            r = i * CUNROLL + u
            for b in range(D // 32):
                w = gbuf[r, pl.ds(16 * b, 16)]
                fbuf[r, pl.ds(32 * b, 16)] = lax.bitcast_convert_type(
                    w << 16, jnp.float32)
                fbuf[r, pl.ds(32 * b + 16, 16)] = lax.bitcast_convert_type(
                    w & mask, jnp.float32)
        return _

    lax.fori_loop(0, CHUNK // CUNROLL, crow, None)


def _sc_aggregate_body(x_hbm, xbf_hbm, src_hbm, dst_hbm, zeros_hbm, agg_hbm,
                       src_v, dst_v, g0, g1, g2, g3, f0, f1, agg_sh,
                       gs0, gs1, gs2, gs3, ss0, ss1):
    cid = lax.axis_index("c")
    sid = lax.axis_index("s")
    wid = cid * NS + sid

    r0 = sid * ROWS_PER_TILE
    pltpu.sync_copy(zeros_hbm, f0.at[pl.ds(0, ZB)])

    @pl.when(cid == 0)
    def _():
        @pl.when(sid < NS - 1)
        def _():
            pltpu.sync_copy(x_hbm.at[pl.ds(r0, ROWS_PER_TILE)],
                            agg_sh.at[pl.ds(r0, ROWS_PER_TILE)])

        @pl.when(sid == NS - 1)
        def _():
            pltpu.sync_copy(x_hbm.at[pl.ds(r0, N - r0)],
                            agg_sh.at[pl.ds(r0, N - r0)])
            pltpu.sync_copy(f0.at[pl.ds(0, ZB)], agg_sh.at[pl.ds(N, ZB)])
            pltpu.sync_copy(f0.at[pl.ds(0, ZB)],
                            agg_sh.at[pl.ds(N + ZB, ZB)])
            pltpu.sync_copy(f0.at[pl.ds(0, NPAD - N - 2 * ZB)],
                            agg_sh.at[pl.ds(N + 2 * ZB, NPAD - N - 2 * ZB)])

    @pl.when(cid == 1)
    def _():
        for t in range(ROWS_PER_TILE // ZB):
            pltpu.sync_copy(f0.at[pl.ds(0, ZB)],
                            agg_sh.at[pl.ds(r0 + t * ZB, ZB)])
        rem = ROWS_PER_TILE % ZB
        pltpu.sync_copy(f0.at[pl.ds(0, rem)],
                        agg_sh.at[pl.ds(r0 + ROWS_PER_TILE - rem, rem)])

    plsc.subcore_barrier()

    gbufs, gsems = (g0, g1, g2, g3), (gs0, gs1, gs2, gs3)
    fbufs, ssems = (f0, f1), (ss0, ss1)

    def gather(j, q):
        pltpu.async_copy(xbf_hbm.at[src_v.at[j]], gbufs[q], gsems[q])

    def gwait(j, q):
        pltpu.make_async_copy(xbf_hbm.at[src_v.at[j]], gbufs[q],
                              gsems[q]).wait()

    def scat(j, p):
        pltpu.async_copy(fbufs[p], agg_sh.at[dst_v.at[j]], ssems[p],
                         add=True)

    def swait(j, p):
        pltpu.make_async_copy(fbufs[p], agg_sh.at[dst_v.at[j]],
                              ssems[p]).wait()

    def block(blk, _unused):
        pltpu.sync_copy(src_hbm.at[wid, blk], src_v)
        pltpu.sync_copy(dst_hbm.at[wid, blk], dst_v)

        gather(0, 0)
        gather(1, 1)
        gather(2, 2)

        gather(3, 3)
        gwait(0, 0)
        _convert_chunk(g0, f0)
        scat(0, 0)
        gather(4, 0)
        gwait(1, 1)
        _convert_chunk(g1, f1)
        scat(1, 1)
        gather(5, 1)
        gwait(2, 2)
        swait(0, 0)
        _convert_chunk(g2, f0)
        scat(2, 0)
        gather(6, 2)
        gwait(3, 3)
        swait(1, 1)
        _convert_chunk(g3, f1)
        scat(3, 1)

        def body(i, _):
            j0 = 4 * i
            gather(j0 + 3, 3)
            gwait(j0, 0)
            swait(j0 - 2, 0)
            _convert_chunk(g0, f0)
            scat(j0, 0)
            gather(j0 + 4, 0)
            gwait(j0 + 1, 1)
            swait(j0 - 1, 1)
            _convert_chunk(g1, f1)
            scat(j0 + 1, 1)
            gather(j0 + 5, 1)
            gwait(j0 + 2, 2)
            swait(j0, 0)
            _convert_chunk(g2, f0)
            scat(j0 + 2, 0)
            gather(j0 + 6, 2)
            gwait(j0 + 3, 3)
            swait(j0 + 1, 1)
            _convert_chunk(g3, f1)
            scat(j0 + 3, 1)
            return _

        lax.fori_loop(1, (BCH - 4) // 4, body, None)

        jl = BCH - 4
        gather(jl + 3, 3)
        gwait(jl, 0)
        swait(jl - 2, 0)
        _convert_chunk(g0, f0)
        scat(jl, 0)
        gwait(jl + 1, 1)
        swait(jl - 1, 1)
        _convert_chunk(g1, f1)
        scat(jl + 1, 1)
        gwait(jl + 2, 2)
        swait(jl, 0)
        _convert_chunk(g2, f0)
        scat(jl + 2, 0)
        gwait(jl + 3, 3)
        swait(jl + 1, 1)
        _convert_chunk(g3, f1)
        scat(jl + 3, 1)
        swait(jl + 2, 0)
        swait(jl + 3, 1)
        return _unused

    lax.fori_loop(0, NBLK, block, None)

    plsc.subcore_barrier()

    pltpu.sync_copy(agg_sh.at[pl.ds(r0, ROWS_PER_TILE)],
                    agg_hbm.at[cid, pl.ds(r0, ROWS_PER_TILE)])


def _sc_aggregate(x, xbf, src3, dst3, zeros):
    mesh = plsc.VectorSubcoreMesh(core_axis_name="c", subcore_axis_name="s")
    fn = functools.partial(
        pl.kernel,
        out_type=jax.ShapeDtypeStruct((NC, NPAD, D), jnp.float32),
        mesh=mesh,
        compiler_params=pltpu.CompilerParams(use_tc_tiling_on_sc=False),
        scratch_types=[
            pltpu.VMEM((BCH, CHUNK), jnp.int32),
            pltpu.VMEM((BCH, CHUNK), jnp.int32),
            pltpu.VMEM((CHUNK, D // 2), jnp.int32),
            pltpu.VMEM((CHUNK, D // 2), jnp.int32),
            pltpu.VMEM((CHUNK, D // 2), jnp.int32),
            pltpu.VMEM((CHUNK, D // 2), jnp.int32),
            pltpu.VMEM((CHUNK, D), jnp.float32),
            pltpu.VMEM((CHUNK, D), jnp.float32),
            pltpu.VMEM_SHARED((NPAD, D), jnp.float32),
            pltpu.SemaphoreType.DMA,
            pltpu.SemaphoreType.DMA,
            pltpu.SemaphoreType.DMA,
            pltpu.SemaphoreType.DMA,
            pltpu.SemaphoreType.DMA,
            pltpu.SemaphoreType.DMA,
        ],
    )(_sc_aggregate_body)
    return fn(x, xbf, src3, dst3, zeros)


def _tc_linear_body(agg_ref, wt_ref, b_ref, out_ref):
    h = agg_ref[0] + agg_ref[1]
    h = jnp.dot(h, wt_ref[...], preferred_element_type=jnp.float32)
    out_ref[...] = jnp.maximum(h + b_ref[...], 0.0)


def _tc_linear(agg, wt, b2):
    bm = 1000
    grid = (N // bm,)
    return pl.pallas_call(
        _tc_linear_body,
        grid=grid,
        in_specs=[
            pl.BlockSpec((NC, bm, D), lambda i: (0, i, 0)),
            pl.BlockSpec((D, D), lambda i: (0, 0)),
            pl.BlockSpec((1, D), lambda i: (0, 0)),
        ],
        out_specs=pl.BlockSpec((bm, D), lambda i: (i, 0)),
        out_shape=jax.ShapeDtypeStruct((N, D), jnp.float32),
    )(agg, wt, b2)


def kernel(x, edge_index, W, b):
    src3 = edge_index[0].astype(jnp.int32).reshape(NW, NBLK, BCH, CHUNK)
    dst3 = edge_index[1].astype(jnp.int32).reshape(NW, NBLK, BCH, CHUNK)
    xbf = lax.bitcast_convert_type(
        x[:, _PERM].astype(jnp.bfloat16).reshape(N, D // 2, 2), jnp.int32)
    zeros = jnp.zeros((ZB, D), jnp.float32)
    agg = _sc_aggregate(x, xbf, src3, dst3, zeros)
    return _tc_linear(agg, W.T, b.reshape(1, D))


# speedup vs baseline: 2.0615x; 2.0615x over previous
import functools

import jax
import jax.numpy as jnp
from jax import lax
from jax.experimental import pallas as pl
from jax.experimental.pallas import tpu as pltpu
from jax.experimental.pallas import tpu_sc as plsc

N = 10000
E = 320000
D = 128

NC = 2
NS = 16
NW = NC * NS

CHUNK = 50
EPT = E // NW
NCHUNK = EPT // CHUNK
BCH = 20
NBLK = NCHUNK // BCH
ZB = 48
ROWS_PER_TILE = 632
NPAD = ROWS_PER_TILE * NS


def _sc_aggregate_body(x_hbm, src_hbm, dst_hbm, zeros_hbm, agg_hbm,
                       src_v0, dst_v0, src_v1, dst_v1,
                       rows0, rows1, rows2, rows3, agg_sh,
                       sem0, sem1, sem2, sem3, semi0, semi1):
    cid = lax.axis_index("c")
    sid = lax.axis_index("s")
    wid = cid * NS + sid

    r0 = sid * ROWS_PER_TILE
    pltpu.async_copy(src_hbm.at[wid, 0], src_v0, semi0)
    pltpu.async_copy(dst_hbm.at[wid, 0], dst_v0, semi0)
    pltpu.sync_copy(zeros_hbm, rows0.at[pl.ds(0, ZB)])

    @pl.when(cid == 0)
    def _():
        @pl.when(sid < NS - 1)
        def _():
            pltpu.sync_copy(x_hbm.at[pl.ds(r0, ROWS_PER_TILE)],
                            agg_sh.at[pl.ds(r0, ROWS_PER_TILE)])

        @pl.when(sid == NS - 1)
        def _():
            pltpu.sync_copy(x_hbm.at[pl.ds(r0, N - r0)],
                            agg_sh.at[pl.ds(r0, N - r0)])
            pltpu.sync_copy(rows0.at[pl.ds(0, ZB)],
                            agg_sh.at[pl.ds(N, ZB)])
            pltpu.sync_copy(rows0.at[pl.ds(0, ZB)],
                            agg_sh.at[pl.ds(N + ZB, ZB)])
            pltpu.sync_copy(rows0.at[pl.ds(0, NPAD - N - 2 * ZB)],
                            agg_sh.at[pl.ds(N + 2 * ZB, NPAD - N - 2 * ZB)])

    @pl.when(cid == 1)
    def _():
        for t in range(ROWS_PER_TILE // ZB):
            pltpu.sync_copy(rows0.at[pl.ds(0, ZB)],
                            agg_sh.at[pl.ds(r0 + t * ZB, ZB)])
        rem = ROWS_PER_TILE % ZB
        pltpu.sync_copy(
            rows0.at[pl.ds(0, rem)],
            agg_sh.at[pl.ds(r0 + ROWS_PER_TILE - rem, rem)])

    plsc.subcore_barrier()

    for blk in range(NBLK):
        if blk % 2 == 0:
            src_v, dst_v, semi = src_v0, dst_v0, semi0
            nsrc_v, ndst_v, nsemi = src_v1, dst_v1, semi1
        else:
            src_v, dst_v, semi = src_v1, dst_v1, semi1
            nsrc_v, ndst_v, nsemi = src_v0, dst_v0, semi0
        pltpu.make_async_copy(src_hbm.at[wid, blk], src_v, semi).wait()
        pltpu.make_async_copy(dst_hbm.at[wid, blk], dst_v, semi).wait()
        if blk + 1 < NBLK:
            pltpu.async_copy(src_hbm.at[wid, blk + 1], nsrc_v, nsemi)
            pltpu.async_copy(dst_hbm.at[wid, blk + 1], ndst_v, nsemi)

        pltpu.async_copy(x_hbm.at[src_v.at[0]], rows0, sem0)
        pltpu.async_copy(x_hbm.at[src_v.at[1]], rows1, sem1)
        pltpu.async_copy(x_hbm.at[src_v.at[2]], rows2, sem2)

        def body(i, _):
            j0 = 4 * i
            pltpu.async_copy(x_hbm.at[src_v.at[j0 + 3]], rows3, sem3)
            pltpu.make_async_copy(x_hbm.at[src_v.at[j0]], rows0, sem0).wait()
            pltpu.sync_copy(rows0, agg_sh.at[dst_v.at[j0]], add=True)
            pltpu.async_copy(x_hbm.at[src_v.at[j0 + 4]], rows0, sem0)
            pltpu.make_async_copy(x_hbm.at[src_v.at[j0 + 1]], rows1,
                                  sem1).wait()
            pltpu.sync_copy(rows1, agg_sh.at[dst_v.at[j0 + 1]], add=True)
            pltpu.async_copy(x_hbm.at[src_v.at[j0 + 5]], rows1, sem1)
            pltpu.make_async_copy(x_hbm.at[src_v.at[j0 + 2]], rows2,
                                  sem2).wait()
            pltpu.sync_copy(rows2, agg_sh.at[dst_v.at[j0 + 2]], add=True)
            pltpu.async_copy(x_hbm.at[src_v.at[j0 + 6]], rows2, sem2)
            pltpu.make_async_copy(x_hbm.at[src_v.at[j0 + 3]], rows3,
                                  sem3).wait()
            pltpu.sync_copy(rows3, agg_sh.at[dst_v.at[j0 + 3]], add=True)
            return _

        lax.fori_loop(0, (BCH - 4) // 4, body, None)

        jlast = BCH - 4
        pltpu.async_copy(x_hbm.at[src_v.at[jlast + 3]], rows3, sem3)
        pltpu.make_async_copy(x_hbm.at[src_v.at[jlast]], rows0, sem0).wait()
        pltpu.sync_copy(rows0, agg_sh.at[dst_v.at[jlast]], add=True)
        pltpu.make_async_copy(x_hbm.at[src_v.at[jlast + 1]], rows1,
                              sem1).wait()
        pltpu.sync_copy(rows1, agg_sh.at[dst_v.at[jlast + 1]], add=True)
        pltpu.make_async_copy(x_hbm.at[src_v.at[jlast + 2]], rows2,
                              sem2).wait()
        pltpu.sync_copy(rows2, agg_sh.at[dst_v.at[jlast + 2]], add=True)
        pltpu.make_async_copy(x_hbm.at[src_v.at[jlast + 3]], rows3,
                              sem3).wait()
        pltpu.sync_copy(rows3, agg_sh.at[dst_v.at[jlast + 3]], add=True)

    plsc.subcore_barrier()

    pltpu.sync_copy(agg_sh.at[pl.ds(r0, ROWS_PER_TILE)],
                    agg_hbm.at[cid, pl.ds(r0, ROWS_PER_TILE)])


def _sc_aggregate(x, src3, dst3, zeros):
    mesh = plsc.VectorSubcoreMesh(core_axis_name="c", subcore_axis_name="s")
    fn = functools.partial(
        pl.kernel,
        out_type=jax.ShapeDtypeStruct((NC, NPAD, D), jnp.float32),
        mesh=mesh,
        scratch_types=[
            pltpu.VMEM((BCH, CHUNK), jnp.int32),
            pltpu.VMEM((BCH, CHUNK), jnp.int32),
            pltpu.VMEM((BCH, CHUNK), jnp.int32),
            pltpu.VMEM((BCH, CHUNK), jnp.int32),
            pltpu.VMEM((CHUNK, D), jnp.float32),
            pltpu.VMEM((CHUNK, D), jnp.float32),
            pltpu.VMEM((CHUNK, D), jnp.float32),
            pltpu.VMEM((CHUNK, D), jnp.float32),
            pltpu.VMEM_SHARED((NPAD, D), jnp.float32),
            pltpu.SemaphoreType.DMA,
            pltpu.SemaphoreType.DMA,
            pltpu.SemaphoreType.DMA,
            pltpu.SemaphoreType.DMA,
            pltpu.SemaphoreType.DMA,
            pltpu.SemaphoreType.DMA,
        ],
    )(_sc_aggregate_body)
    return fn(x, src3, dst3, zeros)


def _tc_linear_body(agg_ref, wt_ref, b_ref, out_ref):
    h = agg_ref[0] + agg_ref[1]
    h = jnp.dot(h, wt_ref[...], preferred_element_type=jnp.float32)
    out_ref[...] = jnp.maximum(h + b_ref[...], 0.0)


def _tc_linear(agg, wt, b2):
    bm = 1000
    grid = (N // bm,)
    return pl.pallas_call(
        _tc_linear_body,
        grid=grid,
        in_specs=[
            pl.BlockSpec((NC, bm, D), lambda i: (0, i, 0)),
            pl.BlockSpec((D, D), lambda i: (0, 0)),
            pl.BlockSpec((1, D), lambda i: (0, 0)),
        ],
        out_specs=pl.BlockSpec((bm, D), lambda i: (i, 0)),
        out_shape=jax.ShapeDtypeStruct((N, D), jnp.float32),
    )(agg, wt, b2)


def kernel(x, edge_index, W, b):
    src3 = edge_index[0].astype(jnp.int32).reshape(NW, NBLK, BCH, CHUNK)
    dst3 = edge_index[1].astype(jnp.int32).reshape(NW, NBLK, BCH, CHUNK)
    zeros = jnp.zeros((ZB, D), jnp.float32)
    agg = _sc_aggregate(x, src3, dst3, zeros)
    return _tc_linear(agg, W.T, b.reshape(1, D))


# speedup vs baseline: 2.0624x; 1.0004x over previous
import functools

import jax
import jax.numpy as jnp
from jax import lax
from jax.experimental import pallas as pl
from jax.experimental.pallas import tpu as pltpu
from jax.experimental.pallas import tpu_sc as plsc

N = 10000
E = 320000
D = 128

NC = 2
NS = 16
NW = NC * NS

CHUNK = 50
EPT = E // NW
NCHUNK = EPT // CHUNK
BCH = 20
NBLK = NCHUNK // BCH
ZB = 48
ROWS_PER_TILE = 632
NPAD = ROWS_PER_TILE * NS


def _sc_aggregate_body(x_hbm, src_hbm, dst_hbm, zeros_hbm, agg_hbm,
                       src_v0, dst_v0, src_v1, dst_v1,
                       rows0, rows1, rows2, rows3, agg_sh,
                       sem0, sem1, sem2, sem3, semi0, semi1):
    cid = lax.axis_index("c")
    sid = lax.axis_index("s")
    wid = cid * NS + sid

    r0 = sid * ROWS_PER_TILE
    pltpu.async_copy(src_hbm.at[wid, 0], src_v0, semi0)
    pltpu.async_copy(dst_hbm.at[wid, 0], dst_v0, semi0)
    pltpu.sync_copy(zeros_hbm, rows0.at[pl.ds(0, ZB)])

    @pl.when(cid == 0)
    def _():
        @pl.when(sid < NS - 1)
        def _():
            pltpu.sync_copy(x_hbm.at[pl.ds(r0, ROWS_PER_TILE)],
                            agg_sh.at[pl.ds(r0, ROWS_PER_TILE)])

        @pl.when(sid == NS - 1)
        def _():
            pltpu.sync_copy(x_hbm.at[pl.ds(r0, N - r0)],
                            agg_sh.at[pl.ds(r0, N - r0)])
            pltpu.sync_copy(rows0.at[pl.ds(0, ZB)],
                            agg_sh.at[pl.ds(N, ZB)])
            pltpu.sync_copy(rows0.at[pl.ds(0, ZB)],
                            agg_sh.at[pl.ds(N + ZB, ZB)])
            pltpu.sync_copy(rows0.at[pl.ds(0, NPAD - N - 2 * ZB)],
                            agg_sh.at[pl.ds(N + 2 * ZB, NPAD - N - 2 * ZB)])

    @pl.when(cid == 1)
    def _():
        for t in range(ROWS_PER_TILE // ZB):
            pltpu.sync_copy(rows0.at[pl.ds(0, ZB)],
                            agg_sh.at[pl.ds(r0 + t * ZB, ZB)])
        rem = ROWS_PER_TILE % ZB
        pltpu.sync_copy(
            rows0.at[pl.ds(0, rem)],
            agg_sh.at[pl.ds(r0 + ROWS_PER_TILE - rem, rem)])

    plsc.subcore_barrier()

    for blk in range(NBLK):
        if blk % 2 == 0:
            src_v, dst_v, semi = src_v0, dst_v0, semi0
            nsrc_v, ndst_v, nsemi = src_v1, dst_v1, semi1
        else:
            src_v, dst_v, semi = src_v1, dst_v1, semi1
            nsrc_v, ndst_v, nsemi = src_v0, dst_v0, semi0
        pltpu.make_async_copy(src_hbm.at[wid, blk], src_v, semi).wait()
        pltpu.make_async_copy(dst_hbm.at[wid, blk], dst_v, semi).wait()
        if blk + 1 < NBLK:
            pltpu.async_copy(src_hbm.at[wid, blk + 1], nsrc_v, nsemi)
            pltpu.async_copy(dst_hbm.at[wid, blk + 1], ndst_v, nsemi)

        pltpu.async_copy(x_hbm.at[src_v.at[0]], rows0, sem0)
        pltpu.async_copy(x_hbm.at[src_v.at[1]], rows1, sem1)
        pltpu.async_copy(x_hbm.at[src_v.at[2]], rows2, sem2)

        def body(i, _):
            j0 = 4 * i
            pltpu.async_copy(x_hbm.at[src_v.at[j0 + 3]], rows3, sem3)
            pltpu.make_async_copy(x_hbm.at[src_v.at[j0]], rows0, sem0).wait()
            pltpu.sync_copy(rows0, agg_sh.at[dst_v.at[j0]], add=True)
            pltpu.async_copy(x_hbm.at[src_v.at[j0 + 4]], rows0, sem0)
            pltpu.make_async_copy(x_hbm.at[src_v.at[j0 + 1]], rows1,
                                  sem1).wait()
            pltpu.sync_copy(rows1, agg_sh.at[dst_v.at[j0 + 1]], add=True)
            pltpu.async_copy(x_hbm.at[src_v.at[j0 + 5]], rows1, sem1)
            pltpu.make_async_copy(x_hbm.at[src_v.at[j0 + 2]], rows2,
                                  sem2).wait()
            pltpu.sync_copy(rows2, agg_sh.at[dst_v.at[j0 + 2]], add=True)
            pltpu.async_copy(x_hbm.at[src_v.at[j0 + 6]], rows2, sem2)
            pltpu.make_async_copy(x_hbm.at[src_v.at[j0 + 3]], rows3,
                                  sem3).wait()
            pltpu.sync_copy(rows3, agg_sh.at[dst_v.at[j0 + 3]], add=True)
            return _

        lax.fori_loop(0, (BCH - 4) // 4, body, None)

        jlast = BCH - 4
        pltpu.async_copy(x_hbm.at[src_v.at[jlast + 3]], rows3, sem3)
        pltpu.make_async_copy(x_hbm.at[src_v.at[jlast]], rows0, sem0).wait()
        pltpu.sync_copy(rows0, agg_sh.at[dst_v.at[jlast]], add=True)
        pltpu.make_async_copy(x_hbm.at[src_v.at[jlast + 1]], rows1,
                              sem1).wait()
        pltpu.sync_copy(rows1, agg_sh.at[dst_v.at[jlast + 1]], add=True)
        pltpu.make_async_copy(x_hbm.at[src_v.at[jlast + 2]], rows2,
                              sem2).wait()
        pltpu.sync_copy(rows2, agg_sh.at[dst_v.at[jlast + 2]], add=True)
        pltpu.make_async_copy(x_hbm.at[src_v.at[jlast + 3]], rows3,
                              sem3).wait()
        pltpu.sync_copy(rows3, agg_sh.at[dst_v.at[jlast + 3]], add=True)

    plsc.subcore_barrier()

    pltpu.sync_copy(agg_sh.at[pl.ds(r0, ROWS_PER_TILE)],
                    agg_hbm.at[cid, pl.ds(r0, ROWS_PER_TILE)])


def _sc_aggregate(x, src3, dst3, zeros):
    mesh = plsc.VectorSubcoreMesh(core_axis_name="c", subcore_axis_name="s")
    fn = functools.partial(
        pl.kernel,
        out_type=jax.ShapeDtypeStruct((NC, NPAD, D), jnp.float32),
        mesh=mesh,
        scratch_types=[
            pltpu.VMEM((BCH, CHUNK), jnp.int32),
            pltpu.VMEM((BCH, CHUNK), jnp.int32),
            pltpu.VMEM((BCH, CHUNK), jnp.int32),
            pltpu.VMEM((BCH, CHUNK), jnp.int32),
            pltpu.VMEM((CHUNK, D), jnp.float32),
            pltpu.VMEM((CHUNK, D), jnp.float32),
            pltpu.VMEM((CHUNK, D), jnp.float32),
            pltpu.VMEM((CHUNK, D), jnp.float32),
            pltpu.VMEM_SHARED((NPAD, D), jnp.float32),
            pltpu.SemaphoreType.DMA,
            pltpu.SemaphoreType.DMA,
            pltpu.SemaphoreType.DMA,
            pltpu.SemaphoreType.DMA,
            pltpu.SemaphoreType.DMA,
            pltpu.SemaphoreType.DMA,
        ],
    )(_sc_aggregate_body)
    return fn(x, src3, dst3, zeros)


def _tc_linear_body(agg_ref, w_ref, b_ref, out_ref):
    h = agg_ref[0] + agg_ref[1]
    h = lax.dot_general(h, w_ref[...], (((1,), (1,)), ((), ())),
                        preferred_element_type=jnp.float32)
    out_ref[...] = jnp.maximum(h + b_ref[...], 0.0)


def _tc_linear(agg, w, b2):
    bm = 1000
    grid = (N // bm,)
    return pl.pallas_call(
        _tc_linear_body,
        grid=grid,
        in_specs=[
            pl.BlockSpec((NC, bm, D), lambda i: (0, i, 0)),
            pl.BlockSpec((D, D), lambda i: (0, 0)),
            pl.BlockSpec((1, D), lambda i: (0, 0)),
        ],
        out_specs=pl.BlockSpec((bm, D), lambda i: (i, 0)),
        out_shape=jax.ShapeDtypeStruct((N, D), jnp.float32),
    )(agg, w, b2)


def kernel(x, edge_index, W, b):
    src3 = edge_index[0].astype(jnp.int32).reshape(NW, NBLK, BCH, CHUNK)
    dst3 = edge_index[1].astype(jnp.int32).reshape(NW, NBLK, BCH, CHUNK)
    zeros = jnp.zeros((ZB, D), jnp.float32)
    agg = _sc_aggregate(x, src3, dst3, zeros)
    return _tc_linear(agg, W, b.reshape(1, D))


# speedup vs baseline: 2.0963x; 1.0164x over previous
import functools

import jax
import jax.numpy as jnp
from jax import lax
from jax.experimental import pallas as pl
from jax.experimental.pallas import tpu as pltpu
from jax.experimental.pallas import tpu_sc as plsc

N = 10000
E = 320000
D = 128

NC = 2
NS = 16
NW = NC * NS

CHUNK = 50
EPT = E // NW
NCHUNK = EPT // CHUNK
BCH = 20
NBLK = NCHUNK // BCH
ZB = 48
ROWS_PER_TILE = 632
NPAD = ROWS_PER_TILE * NS


def _sc_aggregate_body(x_hbm, src_hbm, dst_hbm, zeros_hbm, agg_hbm,
                       src_v0, dst_v0, src_v1, dst_v1,
                       rows0, rows1, rows2, rows3, agg_sh,
                       sem0, sem1, sem2, sem3, semi0, semi1):
    cid = lax.axis_index("c")
    sid = lax.axis_index("s")
    wid = cid * NS + sid

    r0 = sid * ROWS_PER_TILE
    pltpu.async_copy(src_hbm.at[wid, 0], src_v0, semi0)
    pltpu.async_copy(dst_hbm.at[wid, 0], dst_v0, semi0)
    pltpu.sync_copy(zeros_hbm, rows0.at[pl.ds(0, ZB)])

    @pl.when(cid == 0)
    def _():
        @pl.when(sid < NS - 1)
        def _():
            pltpu.sync_copy(x_hbm.at[pl.ds(r0, ROWS_PER_TILE)],
                            agg_sh.at[pl.ds(r0, ROWS_PER_TILE)])

        @pl.when(sid == NS - 1)
        def _():
            pltpu.sync_copy(x_hbm.at[pl.ds(r0, N - r0)],
                            agg_sh.at[pl.ds(r0, N - r0)])
            pltpu.sync_copy(rows0.at[pl.ds(0, ZB)],
                            agg_sh.at[pl.ds(N, ZB)])
            pltpu.sync_copy(rows0.at[pl.ds(0, ZB)],
                            agg_sh.at[pl.ds(N + ZB, ZB)])
            pltpu.sync_copy(rows0.at[pl.ds(0, NPAD - N - 2 * ZB)],
                            agg_sh.at[pl.ds(N + 2 * ZB, NPAD - N - 2 * ZB)])

    @pl.when(cid == 1)
    def _():
        for t in range(ROWS_PER_TILE // ZB):
            pltpu.sync_copy(rows0.at[pl.ds(0, ZB)],
                            agg_sh.at[pl.ds(r0 + t * ZB, ZB)])
        rem = ROWS_PER_TILE % ZB
        pltpu.sync_copy(
            rows0.at[pl.ds(0, rem)],
            agg_sh.at[pl.ds(r0 + ROWS_PER_TILE - rem, rem)])

    plsc.subcore_barrier()

    for blk in range(NBLK):
        if blk % 2 == 0:
            src_v, dst_v, semi = src_v0, dst_v0, semi0
            nsrc_v, ndst_v, nsemi = src_v1, dst_v1, semi1
        else:
            src_v, dst_v, semi = src_v1, dst_v1, semi1
            nsrc_v, ndst_v, nsemi = src_v0, dst_v0, semi0
        pltpu.make_async_copy(src_hbm.at[wid, blk], src_v, semi).wait()
        pltpu.make_async_copy(dst_hbm.at[wid, blk], dst_v, semi).wait()
        if blk + 1 < NBLK:
            pltpu.async_copy(src_hbm.at[wid, blk + 1], nsrc_v, nsemi)
            pltpu.async_copy(dst_hbm.at[wid, blk + 1], ndst_v, nsemi)

        pltpu.async_copy(x_hbm.at[src_v.at[0]], rows0, sem0)
        pltpu.async_copy(x_hbm.at[src_v.at[1]], rows1, sem1)
        pltpu.async_copy(x_hbm.at[src_v.at[2]], rows2, sem2)

        def body(i, _):
            j0 = 4 * i
            pltpu.async_copy(x_hbm.at[src_v.at[j0 + 3]], rows3, sem3)
            pltpu.make_async_copy(x_hbm.at[src_v.at[j0]], rows0, sem0).wait()
            pltpu.sync_copy(rows0, agg_sh.at[dst_v.at[j0]], add=True)
            pltpu.async_copy(x_hbm.at[src_v.at[j0 + 4]], rows0, sem0)
            pltpu.make_async_copy(x_hbm.at[src_v.at[j0 + 1]], rows1,
                                  sem1).wait()
            pltpu.sync_copy(rows1, agg_sh.at[dst_v.at[j0 + 1]], add=True)
            pltpu.async_copy(x_hbm.at[src_v.at[j0 + 5]], rows1, sem1)
            pltpu.make_async_copy(x_hbm.at[src_v.at[j0 + 2]], rows2,
                                  sem2).wait()
            pltpu.sync_copy(rows2, agg_sh.at[dst_v.at[j0 + 2]], add=True)
            pltpu.async_copy(x_hbm.at[src_v.at[j0 + 6]], rows2, sem2)
            pltpu.make_async_copy(x_hbm.at[src_v.at[j0 + 3]], rows3,
                                  sem3).wait()
            pltpu.sync_copy(rows3, agg_sh.at[dst_v.at[j0 + 3]], add=True)
            return _

        lax.fori_loop(0, (BCH - 4) // 4, body, None)

        jlast = BCH - 4
        pltpu.async_copy(x_hbm.at[src_v.at[jlast + 3]], rows3, sem3)
        pltpu.make_async_copy(x_hbm.at[src_v.at[jlast]], rows0, sem0).wait()
        pltpu.sync_copy(rows0, agg_sh.at[dst_v.at[jlast]], add=True)
        pltpu.make_async_copy(x_hbm.at[src_v.at[jlast + 1]], rows1,
                              sem1).wait()
        pltpu.sync_copy(rows1, agg_sh.at[dst_v.at[jlast + 1]], add=True)
        pltpu.make_async_copy(x_hbm.at[src_v.at[jlast + 2]], rows2,
                              sem2).wait()
        pltpu.sync_copy(rows2, agg_sh.at[dst_v.at[jlast + 2]], add=True)
        pltpu.make_async_copy(x_hbm.at[src_v.at[jlast + 3]], rows3,
                              sem3).wait()
        pltpu.sync_copy(rows3, agg_sh.at[dst_v.at[jlast + 3]], add=True)

    plsc.subcore_barrier()

    pltpu.sync_copy(agg_sh.at[pl.ds(r0, ROWS_PER_TILE)],
                    agg_hbm.at[cid, pl.ds(r0, ROWS_PER_TILE)])


def _sc_aggregate(x, src3, dst3, zeros):
    mesh = plsc.VectorSubcoreMesh(core_axis_name="c", subcore_axis_name="s")
    fn = functools.partial(
        pl.kernel,
        out_type=jax.ShapeDtypeStruct((NC, NPAD, D), jnp.float32),
        mesh=mesh,
        scratch_types=[
            pltpu.VMEM((BCH, CHUNK), jnp.int32),
            pltpu.VMEM((BCH, CHUNK), jnp.int32),
            pltpu.VMEM((BCH, CHUNK), jnp.int32),
            pltpu.VMEM((BCH, CHUNK), jnp.int32),
            pltpu.VMEM((CHUNK, D), jnp.float32),
            pltpu.VMEM((CHUNK, D), jnp.float32),
            pltpu.VMEM((CHUNK, D), jnp.float32),
            pltpu.VMEM((CHUNK, D), jnp.float32),
            pltpu.VMEM_SHARED((NPAD, D), jnp.float32),
            pltpu.SemaphoreType.DMA,
            pltpu.SemaphoreType.DMA,
            pltpu.SemaphoreType.DMA,
            pltpu.SemaphoreType.DMA,
            pltpu.SemaphoreType.DMA,
            pltpu.SemaphoreType.DMA,
        ],
    )(_sc_aggregate_body)
    return fn(x, src3, dst3, zeros)


def _tc_linear_body(agg_ref, w_ref, b_ref, out_ref):
    h = agg_ref[0] + agg_ref[1]
    h = lax.dot_general(h, w_ref[...], (((1,), (1,)), ((), ())),
                        preferred_element_type=jnp.float32)
    out_ref[...] = jnp.maximum(h + b_ref[...], 0.0)


def _tc_linear(agg, w, b2):
    bm = 2000
    grid = (N // bm,)
    return pl.pallas_call(
        _tc_linear_body,
        grid=grid,
        in_specs=[
            pl.BlockSpec((NC, bm, D), lambda i: (0, i, 0)),
            pl.BlockSpec((D, D), lambda i: (0, 0)),
            pl.BlockSpec((1, D), lambda i: (0, 0)),
        ],
        out_specs=pl.BlockSpec((bm, D), lambda i: (i, 0)),
        out_shape=jax.ShapeDtypeStruct((N, D), jnp.float32),
    )(agg, w, b2)


def kernel(x, edge_index, W, b):
    src3 = edge_index[0].astype(jnp.int32).reshape(NW, NBLK, BCH, CHUNK)
    dst3 = edge_index[1].astype(jnp.int32).reshape(NW, NBLK, BCH, CHUNK)
    zeros = jnp.zeros((ZB, D), jnp.float32)
    agg = _sc_aggregate(x, src3, dst3, zeros)
    return _tc_linear(agg, W, b.reshape(1, D))
